# baseline (speedup 1.0000x reference)
import functools

import jax
import jax.numpy as jnp
from jax import lax
from jax.experimental import pallas as pl
from jax.experimental.pallas import tpu as pltpu
from jax.experimental.pallas import tpu_sc as plsc

_DIM = 64
_CHUNK = 128


def _build_sc_call(batch, dim):
    info = plsc.get_sparse_core_info()
    nc, ns = info.num_cores, info.num_subcores
    nw = nc * ns
    b_per_w = batch // nw
    n_chunks = b_per_w // _CHUNK
    n_groups = _CHUNK // 16

    mesh = plsc.VectorSubcoreMesh(core_axis_name="c", subcore_axis_name="s")

    @functools.partial(
        pl.kernel,
        mesh=mesh,
        out_type=jax.ShapeDtypeStruct((batch,), jnp.float32),
        compiler_params=pltpu.CompilerParams(
            needs_layout_passes=False, use_tc_tiling_on_sc=False),
        scratch_types=[
            pltpu.VMEM((n_chunks, _CHUNK), jnp.int32),
            pltpu.VMEM((n_chunks, _CHUNK), jnp.int32),
            pltpu.VMEM((n_chunks, _CHUNK), jnp.int32),
            pltpu.VMEM((_CHUNK, dim), jnp.float32),
            pltpu.VMEM((_CHUNK, dim), jnp.float32),
            pltpu.VMEM((_CHUNK, dim), jnp.float32),
            pltpu.VMEM((b_per_w,), jnp.float32),
            pltpu.VMEM((16,), jnp.float32),
            pltpu.VMEM((16,), jnp.float32),
            pltpu.SemaphoreType.DMA,
        ],
    )
    def ukge_sc(hidx_h, ridx_h, tidx_h, ent_h, rel_h, w_h, b_h, out_h,
                hidx_v, ridx_v, tidx_v, h_v, r_v, t_v, out_v, w_v, b_v, sem):
        wid = lax.axis_index("s") * nc + lax.axis_index("c")
        pltpu.sync_copy(hidx_h.at[pl.ds(wid * n_chunks, n_chunks)], hidx_v)
        pltpu.sync_copy(ridx_h.at[pl.ds(wid * n_chunks, n_chunks)], ridx_v)
        pltpu.sync_copy(tidx_h.at[pl.ds(wid * n_chunks, n_chunks)], tidx_v)
        pltpu.sync_copy(w_h, w_v)
        pltpu.sync_copy(b_h, b_v)
        wv = w_v[...]
        bv = b_v[...]

        def chunk_body(c, carry):
            cp_h = pltpu.async_copy(ent_h.at[hidx_v.at[c]], h_v, sem)
            cp_r = pltpu.async_copy(rel_h.at[ridx_v.at[c]], r_v, sem)
            cp_t = pltpu.async_copy(ent_h.at[tidx_v.at[c]], t_v, sem)
            cp_h.wait()
            cp_r.wait()
            cp_t.wait()

            def group_body(g, carry2):
                rows = g * 16 + lax.iota(jnp.int32, 16)
                acc = jnp.zeros((16,), jnp.float32)
                for d in range(dim):
                    dv = jnp.full((16,), d, jnp.int32)
                    acc = acc + (plsc.load_gather(h_v, [rows, dv])
                                 * plsc.load_gather(r_v, [rows, dv])
                                 * plsc.load_gather(t_v, [rows, dv]))
                z = acc * wv + bv
                out_v[pl.ds(c * _CHUNK + g * 16, 16)] = 1.0 / (1.0 + jnp.exp(-z))
                return carry2

            lax.fori_loop(0, n_groups, group_body, 0)
            return carry

        lax.fori_loop(0, n_chunks, chunk_body, 0)
        pltpu.sync_copy(out_v, out_h.at[pl.ds(wid * b_per_w, b_per_w)])

    return ukge_sc


def kernel(x, entity_table, rel_table, lin_w, lin_b):
    batch = x.shape[0]
    xi = x.astype(jnp.int32)
    hidx = xi[:, 0].reshape(batch // _CHUNK, _CHUNK)
    ridx = xi[:, 1].reshape(batch // _CHUNK, _CHUNK)
    tidx = xi[:, 2].reshape(batch // _CHUNK, _CHUNK)
    wvec = jnp.full((16,), lin_w[0, 0], jnp.float32)
    bvec = jnp.full((16,), lin_b[0], jnp.float32)
    call = _build_sc_call(batch, entity_table.shape[1])
    return call(hidx, ridx, tidx, entity_table, rel_table, wvec, bvec)
